# baseline (speedup 1.0000x reference)
import functools

import jax
import jax.numpy as jnp
from jax import lax
from jax.experimental import pallas as pl
from jax.experimental.pallas import tpu as pltpu
from jax.experimental.pallas import tpu_sc as plsc

_B = 1024
_K = 256
_N = 100000
_CHOOSE = 256
_INV_TEMP = 40.0


_NB = 1024
_NPAD = ((_N + _NB - 1) // _NB) * _NB
_SLAB = _NPAD // 128


def _mm_body(q_ref, nq_ref, k_ref, nk_ref, out_ref, qn_ref):
    i = pl.program_id(0)

    @pl.when(i == 0)
    def _():
        qn_ref[...] = q_ref[...] / nq_ref[...]

    kn = k_ref[...] / nk_ref[...]
    for t in range(_NB // 128):
        out_ref[:, t, :] = jax.lax.dot_general(
            qn_ref[...], kn[t * 128:(t + 1) * 128, :],
            (((1,), (1,)), ((), ())),
            preferred_element_type=jnp.float32,
            precision=jax.lax.Precision.DEFAULT)


_sims_call = pl.pallas_call(
    _mm_body,
    grid=(_NPAD // _NB,),
    in_specs=[
        pl.BlockSpec((_B, _K), lambda i: (0, 0)),
        pl.BlockSpec((_B, 1), lambda i: (0, 0)),
        pl.BlockSpec((_NB, _K), lambda i: (i, 0)),
        pl.BlockSpec((_NB, 1), lambda i: (i, 0)),
    ],
    out_specs=pl.BlockSpec((_B, _NB // 128, 128), lambda i: (0, i, 0)),
    out_shape=jax.ShapeDtypeStruct((_B, _SLAB, 128), jnp.float32),
    scratch_shapes=[pltpu.VMEM((_B, _K), jnp.float32)],
)


_L = 16
_NW = 32
_ROWS_PER_W = _B // _NW
_VREGS_ROW = _N // _L
_NBINS = 1024
_LO = -1.03125
_SCALE = 496.0
_CAP = 512
_CVR = _CAP // _L


def _sc_body(sims_hbm, value_hbm, out_sims, out_idx, out_y,
             row_v, hist_v, binsum_v, cand_val, cand_idx, osims_v, oidx_v,
             vals_v, ybuf, sem, sem_g):
    wid = lax.axis_index("s") * 2 + lax.axis_index("c")
    lanes = lax.iota(jnp.int32, _L)
    lane_base = lanes * (_NBINS + 1)
    ones_i = jnp.ones((_L,), jnp.int32)
    zeros_i = jnp.zeros((_L,), jnp.int32)
    neg2 = jnp.full((_L,), -2.0, jnp.float32)

    def rload(v):
        s = v // 8
        return row_v[s, pl.ds((v - s * 8) * _L, _L)]

    def cval(v):
        return cand_val[pl.ds(v * _L, _L)]

    def cidx(v):
        return cand_idx[pl.ds(v * _L, _L)]

    pltpu.async_copy(sims_hbm.at[wid * _ROWS_PER_W], row_v, sem)

    @pl.loop(0, _ROWS_PER_W)
    def _row(rl):
        r = wid * _ROWS_PER_W + rl
        pltpu.make_async_copy(sims_hbm.at[r], row_v, sem).wait()

        @pl.loop(0, _NBINS + 1, unroll=8)
        def _zh(i):
            hist_v[pl.ds(i * _L, _L)] = zeros_i

        @pl.loop(0, _VREGS_ROW, unroll=10)
        def _pa(j):
            v = rload(j)
            b = (v * _SCALE + (0.5 - _LO * _SCALE)).astype(jnp.int32)
            plsc.addupdate_scatter(hist_v, [lane_base + b], ones_i)

        @pl.loop(0, _NBINS // _L, unroll=2)
        def _bs(c):
            acc = hist_v[pl.ds(c * _L, _L)]
            for l in range(1, _L):
                acc = acc + hist_v[pl.ds(l * (_NBINS + 1) + c * _L, _L)]
            binsum_v[pl.ds(c * _L, _L)] = acc

        def _chunk_tot(c):
            return jnp.sum(binsum_v[pl.ds(c * _L, _L)])

        def _wcond(carry):
            cum, c = carry
            return jnp.logical_and(c > 0, cum + _chunk_tot(c) < _CHOOSE)

        def _wstep(carry):
            cum, c = carry
            return cum + _chunk_tot(c), c - 1

        cum, cstar = lax.while_loop(
            _wcond, _wstep, (jnp.int32(0), jnp.int32(_NBINS // _L - 1)))
        sfx = cum + plsc.cumsum(lax.rev(binsum_v[pl.ds(cstar * _L, _L)], (0,)))
        i = jnp.max(plsc.all_reduce_ffs(sfx >= _CHOOSE))
        bstar = cstar * _L + (_L - 1) - i
        tf = (bstar.astype(jnp.float32) - jnp.float32(0.5 - _LO * _SCALE)
              - 0.5) * jnp.float32(1.0 / _SCALE)
        tfv = jnp.broadcast_to(tf, (_L,))

        @pl.loop(0, _CVR)
        def _zc(i):
            cand_val[pl.ds(i * _L, _L)] = neg2
            cand_idx[pl.ds(i * _L, _L)] = zeros_i

        _G = 5

        def _pb(g, carry):
            off, jvec = carry
            vs = [rload(g * _G + k) for k in range(_G)]
            ms = [v >= tfv for v in vs]
            many = ms[0]
            for k in range(1, _G):
                many = jnp.logical_or(many, ms[k])

            def _scatter():
                o = off
                for k in range(_G):
                    pos = o + plsc.cumsum(ms[k].astype(jnp.int32)) - 1
                    gk = jnp.logical_and(ms[k], pos < _CAP)
                    plsc.store_scatter(cand_val, [pos], vs[k], mask=gk)
                    plsc.store_scatter(cand_idx, [pos], jvec + k * _L, mask=gk)
                    o = o + plsc.all_reduce_population_count(ms[k])
                return o

            new_off = lax.cond(jnp.any(many), _scatter, lambda: off)
            return new_off, jvec + _G * _L

        lax.fori_loop(0, _VREGS_ROW // _G, _pb, (zeros_i, lanes))

        @pl.when(rl + 1 < _ROWS_PER_W)
        def _pref():
            pltpu.async_copy(sims_hbm.at[r + 1], row_v, sem)

        def _ce(a, b, kv):
            ka = cval(a)
            kb = cval(b)
            ia = cidx(a)
            ib = cidx(b)
            desc = jnp.broadcast_to((a & kv) == 0, (_L,))
            swap = jnp.where(desc, ka < kb, ka > kb)
            cand_val[pl.ds(a * _L, _L)] = jnp.where(swap, kb, ka)
            cand_val[pl.ds(b * _L, _L)] = jnp.where(swap, ka, kb)
            cand_idx[pl.ds(a * _L, _L)] = jnp.where(swap, ib, ia)
            cand_idx[pl.ds(b * _L, _L)] = jnp.where(swap, ia, ib)

        def _vsort(v, desc):
            ks, xs = plsc.sort_key_val(cval(v), cidx(v), descending=desc)
            cand_val[pl.ds(v * _L, _L)] = ks
            cand_idx[pl.ds(v * _L, _L)] = xs

        @pl.loop(0, _CVR // 2)
        def _base(t):
            _vsort(2 * t, True)
            _vsort(2 * t + 1, False)

        for kv in (2, 4, 8, 16, 32):
            jv = kv // 2
            while jv >= 1:
                @pl.loop(0, _CVR // 2)
                def _stage(t, jv=jv, kv=kv):
                    blk = t // jv
                    a = blk * (2 * jv) + (t - blk * jv)
                    _ce(a, a + jv, kv)
                jv //= 2
            if kv < _CVR:
                @pl.loop(0, _CVR // 2)
                def _resort(t, kv=kv):
                    blk = t // kv
                    v = blk * (2 * kv) + (t - blk * kv)
                    _vsort(v, True)
                    _vsort(v + kv, False)
            else:
                @pl.loop(0, _CVR)
                def _resort_all(v):
                    _vsort(v, True)

        @pl.loop(0, _CHOOSE // _L)
        def _st(t):
            s = t // 8
            c = (t - s * 8) * _L
            osims_v[s, pl.ds(c, _L)] = cval(t)
            oidx_v[s, pl.ds(c, _L)] = cidx(t)

        cp0 = pltpu.async_copy(value_hbm.at[oidx_v.at[0]], vals_v.at[0], sem_g)
        cp1 = pltpu.async_copy(value_hbm.at[oidx_v.at[1]], vals_v.at[1], sem_g)
        pltpu.sync_copy(osims_v, out_sims.at[r])
        pltpu.sync_copy(oidx_v, out_idx.at[r])
        cp0.wait()
        cp1.wait()

        mx = jnp.max(cval(0))

        def _sm(t, carry):
            accn, accd = carry
            s = t // 8
            c = (t - s * 8) * _L
            e = jnp.exp((osims_v[s, pl.ds(c, _L)] - mx) * _INV_TEMP)
            return accn + e * vals_v[s, pl.ds(c, _L)], accd + e

        accn, accd = lax.fori_loop(
            0, _CHOOSE // _L, _sm,
            (jnp.zeros((_L,), jnp.float32), jnp.zeros((_L,), jnp.float32)))
        yv = (jnp.broadcast_to(jnp.sum(accn), (_L,))
              / jnp.broadcast_to(jnp.sum(accd), (_L,)))
        plsc.store_scatter(ybuf, [jnp.broadcast_to(rl, (_L,))],
                           yv, mask=lanes < 1)

    pltpu.sync_copy(ybuf, out_y.at[pl.ds(wid * _ROWS_PER_W, _ROWS_PER_W)])


_sc_select = pl.kernel(
    _sc_body,
    out_type=(
        jax.ShapeDtypeStruct((_B, 2, 128), jnp.float32),
        jax.ShapeDtypeStruct((_B, 2, 128), jnp.int32),
        jax.ShapeDtypeStruct((_B,), jnp.float32),
    ),
    mesh=plsc.VectorSubcoreMesh(core_axis_name="c", subcore_axis_name="s"),
    compiler_params=pltpu.CompilerParams(needs_layout_passes=False),
    scratch_types=[
        pltpu.VMEM((_SLAB, 128), jnp.float32),
        pltpu.VMEM(((_NBINS + 1) * _L,), jnp.int32),
        pltpu.VMEM((_NBINS,), jnp.int32),
        pltpu.VMEM((_CAP,), jnp.float32),
        pltpu.VMEM((_CAP,), jnp.int32),
        pltpu.VMEM((2, 128), jnp.float32),
        pltpu.VMEM((2, 128), jnp.int32),
        pltpu.VMEM((2, 128), jnp.float32),
        pltpu.VMEM((_ROWS_PER_W,), jnp.float32),
        pltpu.SemaphoreType.DMA,
        pltpu.SemaphoreType.DMA,
    ],
)


def kernel(input, keys, value):
    nq = jnp.linalg.norm(input, axis=-1, keepdims=True) + 1e-8
    nk = jnp.linalg.norm(keys, axis=-1, keepdims=True) + 1e-8
    keys_pad = jnp.pad(keys, ((0, _NPAD - _N), (0, 0)))
    nk_pad = jnp.pad(nk, ((0, _NPAD - _N), (0, 0)), constant_values=1.0)
    sims = _sims_call(input, nq, keys_pad, nk_pad)
    topk_sims, topk_idx, y = _sc_select(sims, value)
    return (y, topk_sims.reshape(_B, _CHOOSE), topk_idx.reshape(_B, _CHOOSE))


# speedup vs baseline: 1.0001x; 1.0001x over previous
import functools

import jax
import jax.numpy as jnp
from jax import lax
from jax.experimental import pallas as pl
from jax.experimental.pallas import tpu as pltpu
from jax.experimental.pallas import tpu_sc as plsc

_B = 1024
_K = 256
_N = 100000
_CHOOSE = 256
_INV_TEMP = 40.0


_NB = 1024
_NPAD = ((_N + _NB - 1) // _NB) * _NB
_SLAB = _NPAD // 128


def _mm_body(q_ref, nq_ref, k_ref, nk_ref, out_ref, qn_ref):
    i = pl.program_id(0)

    @pl.when(i == 0)
    def _():
        qn_ref[...] = q_ref[...] / nq_ref[...]

    kn = k_ref[...] / nk_ref[...]
    for t in range(_NB // 128):
        out_ref[:, t, :] = jax.lax.dot_general(
            qn_ref[...], kn[t * 128:(t + 1) * 128, :],
            (((1,), (1,)), ((), ())),
            preferred_element_type=jnp.float32,
            precision=jax.lax.Precision.DEFAULT)


_sims_call = pl.pallas_call(
    _mm_body,
    grid=(_NPAD // _NB,),
    in_specs=[
        pl.BlockSpec((_B, _K), lambda i: (0, 0)),
        pl.BlockSpec((_B, 1), lambda i: (0, 0)),
        pl.BlockSpec((_NB, _K), lambda i: (i, 0)),
        pl.BlockSpec((_NB, 1), lambda i: (i, 0)),
    ],
    out_specs=pl.BlockSpec((_B, _NB // 128, 128), lambda i: (0, i, 0)),
    out_shape=jax.ShapeDtypeStruct((_B, _SLAB, 128), jnp.float32),
    scratch_shapes=[pltpu.VMEM((_B, _K), jnp.float32)],
)


_L = 16
_NW = 32
_ROWS_PER_W = _B // _NW
_VREGS_ROW = _N // _L
_NBINS = 1024
_LO = -1.03125
_SCALE = 496.0
_CAP = 512
_CVR = _CAP // _L


def _sc_body(sims_hbm, value_hbm, out_sims, out_idx, out_y,
             row_v, hist_v, binsum_v, cand_val, cand_idx, osims_v, oidx_v,
             vals_v, ybuf, sem, sem_g):
    wid = lax.axis_index("s") * 2 + lax.axis_index("c")
    lanes = lax.iota(jnp.int32, _L)
    lane_base = lanes * (_NBINS + 1)
    ones_i = jnp.ones((_L,), jnp.int32)
    zeros_i = jnp.zeros((_L,), jnp.int32)
    neg2 = jnp.full((_L,), -2.0, jnp.float32)

    def rload(v):
        s = v // 8
        return row_v[s, pl.ds((v - s * 8) * _L, _L)]

    def cval(v):
        return cand_val[pl.ds(v * _L, _L)]

    def cidx(v):
        return cand_idx[pl.ds(v * _L, _L)]

    pltpu.async_copy(sims_hbm.at[wid * _ROWS_PER_W], row_v, sem)

    @pl.loop(0, _ROWS_PER_W)
    def _row(rl):
        r = wid * _ROWS_PER_W + rl
        pltpu.make_async_copy(sims_hbm.at[r], row_v, sem).wait()

        sc0 = jax.named_scope("phA")
        sc0.__enter__()
        @pl.loop(0, _NBINS + 1, unroll=8)
        def _zh(i):
            hist_v[pl.ds(i * _L, _L)] = zeros_i

        @pl.loop(0, _VREGS_ROW, unroll=10)
        def _pa(j):
            v = rload(j)
            b = (v * _SCALE + (0.5 - _LO * _SCALE)).astype(jnp.int32)
            plsc.addupdate_scatter(hist_v, [lane_base + b], ones_i)

        sc0.__exit__(None, None, None)
        sc1 = jax.named_scope("phScan")
        sc1.__enter__()
        @pl.loop(0, _NBINS // _L, unroll=2)
        def _bs(c):
            acc = hist_v[pl.ds(c * _L, _L)]
            for l in range(1, _L):
                acc = acc + hist_v[pl.ds(l * (_NBINS + 1) + c * _L, _L)]
            binsum_v[pl.ds(c * _L, _L)] = acc

        def _chunk_tot(c):
            return jnp.sum(binsum_v[pl.ds(c * _L, _L)])

        def _wcond(carry):
            cum, c = carry
            return jnp.logical_and(c > 0, cum + _chunk_tot(c) < _CHOOSE)

        def _wstep(carry):
            cum, c = carry
            return cum + _chunk_tot(c), c - 1

        cum, cstar = lax.while_loop(
            _wcond, _wstep, (jnp.int32(0), jnp.int32(_NBINS // _L - 1)))
        sfx = cum + plsc.cumsum(lax.rev(binsum_v[pl.ds(cstar * _L, _L)], (0,)))
        i = jnp.max(plsc.all_reduce_ffs(sfx >= _CHOOSE))
        bstar = cstar * _L + (_L - 1) - i
        tf = (bstar.astype(jnp.float32) - jnp.float32(0.5 - _LO * _SCALE)
              - 0.5) * jnp.float32(1.0 / _SCALE)
        tfv = jnp.broadcast_to(tf, (_L,))

        sc1.__exit__(None, None, None)
        sc2 = jax.named_scope("phB")
        sc2.__enter__()
        @pl.loop(0, _CVR)
        def _zc(i):
            cand_val[pl.ds(i * _L, _L)] = neg2
            cand_idx[pl.ds(i * _L, _L)] = zeros_i

        _G = 5

        def _pb(g, carry):
            off, jvec = carry
            vs = [rload(g * _G + k) for k in range(_G)]
            ms = [v >= tfv for v in vs]
            many = ms[0]
            for k in range(1, _G):
                many = jnp.logical_or(many, ms[k])

            def _scatter():
                o = off
                for k in range(_G):
                    pos = o + plsc.cumsum(ms[k].astype(jnp.int32)) - 1
                    gk = jnp.logical_and(ms[k], pos < _CAP)
                    plsc.store_scatter(cand_val, [pos], vs[k], mask=gk)
                    plsc.store_scatter(cand_idx, [pos], jvec + k * _L, mask=gk)
                    o = o + plsc.all_reduce_population_count(ms[k])
                return o

            new_off = lax.cond(jnp.any(many), _scatter, lambda: off)
            return new_off, jvec + _G * _L

        lax.fori_loop(0, _VREGS_ROW // _G, _pb, (zeros_i, lanes))

        sc2.__exit__(None, None, None)
        sc3 = jax.named_scope("phSort")
        sc3.__enter__()
        @pl.when(rl + 1 < _ROWS_PER_W)
        def _pref():
            pltpu.async_copy(sims_hbm.at[r + 1], row_v, sem)

        def _ce(a, b, kv):
            ka = cval(a)
            kb = cval(b)
            ia = cidx(a)
            ib = cidx(b)
            desc = jnp.broadcast_to((a & kv) == 0, (_L,))
            swap = jnp.where(desc, ka < kb, ka > kb)
            cand_val[pl.ds(a * _L, _L)] = jnp.where(swap, kb, ka)
            cand_val[pl.ds(b * _L, _L)] = jnp.where(swap, ka, kb)
            cand_idx[pl.ds(a * _L, _L)] = jnp.where(swap, ib, ia)
            cand_idx[pl.ds(b * _L, _L)] = jnp.where(swap, ia, ib)

        def _vsort(v, desc):
            ks, xs = plsc.sort_key_val(cval(v), cidx(v), descending=desc)
            cand_val[pl.ds(v * _L, _L)] = ks
            cand_idx[pl.ds(v * _L, _L)] = xs

        @pl.loop(0, _CVR // 2)
        def _base(t):
            _vsort(2 * t, True)
            _vsort(2 * t + 1, False)

        for kv in (2, 4, 8, 16, 32):
            jv = kv // 2
            while jv >= 1:
                @pl.loop(0, _CVR // 2)
                def _stage(t, jv=jv, kv=kv):
                    blk = t // jv
                    a = blk * (2 * jv) + (t - blk * jv)
                    _ce(a, a + jv, kv)
                jv //= 2
            if kv < _CVR:
                @pl.loop(0, _CVR // 2)
                def _resort(t, kv=kv):
                    blk = t // kv
                    v = blk * (2 * kv) + (t - blk * kv)
                    _vsort(v, True)
                    _vsort(v + kv, False)
            else:
                @pl.loop(0, _CVR)
                def _resort_all(v):
                    _vsort(v, True)

        @pl.loop(0, _CHOOSE // _L)
        def _st(t):
            s = t // 8
            c = (t - s * 8) * _L
            osims_v[s, pl.ds(c, _L)] = cval(t)
            oidx_v[s, pl.ds(c, _L)] = cidx(t)

        sc3.__exit__(None, None, None)
        sc4 = jax.named_scope("phOut")
        sc4.__enter__()
        cp0 = pltpu.async_copy(value_hbm.at[oidx_v.at[0]], vals_v.at[0], sem_g)
        cp1 = pltpu.async_copy(value_hbm.at[oidx_v.at[1]], vals_v.at[1], sem_g)
        pltpu.sync_copy(osims_v, out_sims.at[r])
        pltpu.sync_copy(oidx_v, out_idx.at[r])
        cp0.wait()
        cp1.wait()

        mx = jnp.max(cval(0))

        def _sm(t, carry):
            accn, accd = carry
            s = t // 8
            c = (t - s * 8) * _L
            e = jnp.exp((osims_v[s, pl.ds(c, _L)] - mx) * _INV_TEMP)
            return accn + e * vals_v[s, pl.ds(c, _L)], accd + e

        accn, accd = lax.fori_loop(
            0, _CHOOSE // _L, _sm,
            (jnp.zeros((_L,), jnp.float32), jnp.zeros((_L,), jnp.float32)))
        yv = (jnp.broadcast_to(jnp.sum(accn), (_L,))
              / jnp.broadcast_to(jnp.sum(accd), (_L,)))
        plsc.store_scatter(ybuf, [jnp.broadcast_to(rl, (_L,))],
                           yv, mask=lanes < 1)

        sc4.__exit__(None, None, None)

    pltpu.sync_copy(ybuf, out_y.at[pl.ds(wid * _ROWS_PER_W, _ROWS_PER_W)])


_sc_select = pl.kernel(
    _sc_body,
    out_type=(
        jax.ShapeDtypeStruct((_B, 2, 128), jnp.float32),
        jax.ShapeDtypeStruct((_B, 2, 128), jnp.int32),
        jax.ShapeDtypeStruct((_B,), jnp.float32),
    ),
    mesh=plsc.VectorSubcoreMesh(core_axis_name="c", subcore_axis_name="s"),
    compiler_params=pltpu.CompilerParams(needs_layout_passes=False),
    scratch_types=[
        pltpu.VMEM((_SLAB, 128), jnp.float32),
        pltpu.VMEM(((_NBINS + 1) * _L,), jnp.int32),
        pltpu.VMEM((_NBINS,), jnp.int32),
        pltpu.VMEM((_CAP,), jnp.float32),
        pltpu.VMEM((_CAP,), jnp.int32),
        pltpu.VMEM((2, 128), jnp.float32),
        pltpu.VMEM((2, 128), jnp.int32),
        pltpu.VMEM((2, 128), jnp.float32),
        pltpu.VMEM((_ROWS_PER_W,), jnp.float32),
        pltpu.SemaphoreType.DMA,
        pltpu.SemaphoreType.DMA,
    ],
)


def kernel(input, keys, value):
    nq = jnp.linalg.norm(input, axis=-1, keepdims=True) + 1e-8
    nk = jnp.linalg.norm(keys, axis=-1, keepdims=True) + 1e-8
    keys_pad = jnp.pad(keys, ((0, _NPAD - _N), (0, 0)))
    nk_pad = jnp.pad(nk, ((0, _NPAD - _N), (0, 0)), constant_values=1.0)
    sims = _sims_call(input, nq, keys_pad, nk_pad)
    topk_sims, topk_idx, y = _sc_select(sims, value)
    return (y, topk_sims.reshape(_B, _CHOOSE), topk_idx.reshape(_B, _CHOOSE))


# speedup vs baseline: 3.0274x; 3.0271x over previous
import functools

import jax
import jax.numpy as jnp
from jax import lax
from jax.experimental import pallas as pl
from jax.experimental.pallas import tpu as pltpu
from jax.experimental.pallas import tpu_sc as plsc

_B = 1024
_K = 256
_N = 100000
_CHOOSE = 256
_INV_TEMP = 40.0


_NB = 1024
_NPAD = ((_N + _NB - 1) // _NB) * _NB
_SLAB = _NPAD // 128


def _mm_body(q_ref, nq_ref, k_ref, nk_ref, out_ref, qn_ref):
    i = pl.program_id(0)

    @pl.when(i == 0)
    def _():
        qn_ref[...] = q_ref[...] / nq_ref[...]

    kn = k_ref[...] / nk_ref[...]
    for t in range(_NB // 128):
        out_ref[:, t, :] = jax.lax.dot_general(
            qn_ref[...], kn[t * 128:(t + 1) * 128, :],
            (((1,), (1,)), ((), ())),
            preferred_element_type=jnp.float32,
            precision=jax.lax.Precision.DEFAULT)


_sims_call = pl.pallas_call(
    _mm_body,
    grid=(_NPAD // _NB,),
    in_specs=[
        pl.BlockSpec((_B, _K), lambda i: (0, 0)),
        pl.BlockSpec((_B, 1), lambda i: (0, 0)),
        pl.BlockSpec((_NB, _K), lambda i: (i, 0)),
        pl.BlockSpec((_NB, 1), lambda i: (i, 0)),
    ],
    out_specs=pl.BlockSpec((_B, _NB // 128, 128), lambda i: (0, i, 0)),
    out_shape=jax.ShapeDtypeStruct((_B, _SLAB, 128), jnp.float32),
    scratch_shapes=[pltpu.VMEM((_B, _K), jnp.float32)],
)


_L = 16
_NW = 32
_ROWS_PER_W = _B // _NW
_VREGS_ROW = _N // _L
_NBINS = 1024
_LO = -1.03125
_SCALE = 496.0
_CAP = 1024
_PLANE = _CAP // _L
_CVR = _CAP // _L


def _sc_body(sims_hbm, value_hbm, out_sims, out_idx, out_y,
             row_v, hist_v, binsum_v, cand_val, cand_idx, osims_v, oidx_v,
             vals_v, ybuf, sem, sem_g):
    wid = lax.axis_index("s") * 2 + lax.axis_index("c")
    lanes = lax.iota(jnp.int32, _L)
    lane_base = lanes * (_NBINS + 1)
    ones_i = jnp.ones((_L,), jnp.int32)
    zeros_i = jnp.zeros((_L,), jnp.int32)
    neg2 = jnp.full((_L,), -2.0, jnp.float32)

    def rload(v):
        s = v // 8
        return row_v[s, pl.ds((v - s * 8) * _L, _L)]

    def cval(v):
        return cand_val[pl.ds(v * _L, _L)]

    def cidx(v):
        return cand_idx[pl.ds(v * _L, _L)]

    pltpu.async_copy(sims_hbm.at[wid * _ROWS_PER_W], row_v, sem)

    @pl.loop(0, _ROWS_PER_W)
    def _row(rl):
        r = wid * _ROWS_PER_W + rl
        pltpu.make_async_copy(sims_hbm.at[r], row_v, sem).wait()

        sc0 = jax.named_scope("phA")
        sc0.__enter__()
        @plsc.parallel_loop(0, _NBINS + 1, unroll=8)
        def _zh(i):
            hist_v[pl.ds(i * _L, _L)] = zeros_i

        @plsc.parallel_loop(0, _VREGS_ROW, unroll=10)
        def _pa(j):
            v = rload(j)
            b = (v * _SCALE + (0.5 - _LO * _SCALE)).astype(jnp.int32)
            plsc.addupdate_scatter(hist_v, [lane_base + b], ones_i)

        sc0.__exit__(None, None, None)
        sc1 = jax.named_scope("phScan")
        sc1.__enter__()
        @pl.loop(0, _NBINS // _L, unroll=2)
        def _bs(c):
            acc = hist_v[pl.ds(c * _L, _L)]
            for l in range(1, _L):
                acc = acc + hist_v[pl.ds(l * (_NBINS + 1) + c * _L, _L)]
            binsum_v[pl.ds(c * _L, _L)] = acc

        def _chunk_tot(c):
            return jnp.sum(binsum_v[pl.ds(c * _L, _L)])

        def _wcond(carry):
            cum, c = carry
            return jnp.logical_and(c > 0, cum + _chunk_tot(c) < _CHOOSE)

        def _wstep(carry):
            cum, c = carry
            return cum + _chunk_tot(c), c - 1

        cum, cstar = lax.while_loop(
            _wcond, _wstep, (jnp.int32(0), jnp.int32(_NBINS // _L - 1)))
        sfx = cum + plsc.cumsum(lax.rev(binsum_v[pl.ds(cstar * _L, _L)], (0,)))
        i = jnp.max(plsc.all_reduce_ffs(sfx >= _CHOOSE))
        bstar = cstar * _L + (_L - 1) - i
        tf = (bstar.astype(jnp.float32) - jnp.float32(0.5 - _LO * _SCALE)
              - 0.5) * jnp.float32(1.0 / _SCALE)
        tfv = jnp.broadcast_to(tf, (_L,))

        sc1.__exit__(None, None, None)
        sc2 = jax.named_scope("phB")
        sc2.__enter__()
        @plsc.parallel_loop(0, _CVR, unroll=8)
        def _zc(i):
            cand_val[pl.ds(i * _L, _L)] = neg2
            cand_idx[pl.ds(i * _L, _L)] = zeros_i

        laneoff = lanes * _PLANE
        plane_v = jnp.full((_L,), _PLANE, jnp.int32)

        @plsc.parallel_loop(0, _VREGS_ROW, unroll=8,
                            carry=(zeros_i, lanes))
        def _pb(j, carry):
            cnt, jvec = carry
            v = rload(j)
            m = v >= tfv
            g = jnp.logical_and(m, cnt < plane_v)
            pos = laneoff + cnt
            plsc.store_scatter(cand_val, [pos], v, mask=g)
            plsc.store_scatter(cand_idx, [pos], jvec, mask=g)
            return cnt + m.astype(jnp.int32), jvec + _L

        sc2.__exit__(None, None, None)
        sc3 = jax.named_scope("phSort")
        sc3.__enter__()
        @pl.when(rl + 1 < _ROWS_PER_W)
        def _pref():
            pltpu.async_copy(sims_hbm.at[r + 1], row_v, sem)

        def _ce(a, b, kv):
            ka = cval(a)
            kb = cval(b)
            ia = cidx(a)
            ib = cidx(b)
            desc = jnp.broadcast_to((a & kv) == 0, (_L,))
            swap = jnp.where(desc, ka < kb, ka > kb)
            cand_val[pl.ds(a * _L, _L)] = jnp.where(swap, kb, ka)
            cand_val[pl.ds(b * _L, _L)] = jnp.where(swap, ka, kb)
            cand_idx[pl.ds(a * _L, _L)] = jnp.where(swap, ib, ia)
            cand_idx[pl.ds(b * _L, _L)] = jnp.where(swap, ia, ib)

        def _vsort(v, desc):
            ks, xs = plsc.sort_key_val(cval(v), cidx(v), descending=desc)
            cand_val[pl.ds(v * _L, _L)] = ks
            cand_idx[pl.ds(v * _L, _L)] = xs

        @pl.loop(0, _CVR // 2)
        def _base(t):
            _vsort(2 * t, True)
            _vsort(2 * t + 1, False)

        kvs = []
        kv = 2
        while kv <= _CVR:
            kvs.append(kv)
            kv *= 2
        for kv in kvs:
            jv = kv // 2
            while jv >= 1:
                @pl.loop(0, _CVR // 2)
                def _stage(t, jv=jv, kv=kv):
                    blk = t // jv
                    a = blk * (2 * jv) + (t - blk * jv)
                    _ce(a, a + jv, kv)
                jv //= 2
            if kv < _CVR:
                @pl.loop(0, _CVR // 2)
                def _resort(t, kv=kv):
                    blk = t // kv
                    v = blk * (2 * kv) + (t - blk * kv)
                    _vsort(v, True)
                    _vsort(v + kv, False)
            else:
                @pl.loop(0, _CVR)
                def _resort_all(v):
                    _vsort(v, True)

        @pl.loop(0, _CHOOSE // _L)
        def _st(t):
            s = t // 8
            c = (t - s * 8) * _L
            osims_v[s, pl.ds(c, _L)] = cval(t)
            oidx_v[s, pl.ds(c, _L)] = cidx(t)

        sc3.__exit__(None, None, None)
        sc4 = jax.named_scope("phOut")
        sc4.__enter__()
        cp0 = pltpu.async_copy(value_hbm.at[oidx_v.at[0]], vals_v.at[0], sem_g)
        cp1 = pltpu.async_copy(value_hbm.at[oidx_v.at[1]], vals_v.at[1], sem_g)
        pltpu.sync_copy(osims_v, out_sims.at[r])
        pltpu.sync_copy(oidx_v, out_idx.at[r])
        cp0.wait()
        cp1.wait()

        mx = jnp.max(cval(0))

        def _sm(t, carry):
            accn, accd = carry
            s = t // 8
            c = (t - s * 8) * _L
            e = jnp.exp((osims_v[s, pl.ds(c, _L)] - mx) * _INV_TEMP)
            return accn + e * vals_v[s, pl.ds(c, _L)], accd + e

        accn, accd = lax.fori_loop(
            0, _CHOOSE // _L, _sm,
            (jnp.zeros((_L,), jnp.float32), jnp.zeros((_L,), jnp.float32)))
        yv = (jnp.broadcast_to(jnp.sum(accn), (_L,))
              / jnp.broadcast_to(jnp.sum(accd), (_L,)))
        plsc.store_scatter(ybuf, [jnp.broadcast_to(rl, (_L,))],
                           yv, mask=lanes < 1)

        sc4.__exit__(None, None, None)

    pltpu.sync_copy(ybuf, out_y.at[pl.ds(wid * _ROWS_PER_W, _ROWS_PER_W)])


_sc_select = pl.kernel(
    _sc_body,
    out_type=(
        jax.ShapeDtypeStruct((_B, 2, 128), jnp.float32),
        jax.ShapeDtypeStruct((_B, 2, 128), jnp.int32),
        jax.ShapeDtypeStruct((_B,), jnp.float32),
    ),
    mesh=plsc.VectorSubcoreMesh(core_axis_name="c", subcore_axis_name="s"),
    compiler_params=pltpu.CompilerParams(needs_layout_passes=False),
    scratch_types=[
        pltpu.VMEM((_SLAB, 128), jnp.float32),
        pltpu.VMEM(((_NBINS + 1) * _L,), jnp.int32),
        pltpu.VMEM((_NBINS,), jnp.int32),
        pltpu.VMEM((_CAP,), jnp.float32),
        pltpu.VMEM((_CAP,), jnp.int32),
        pltpu.VMEM((2, 128), jnp.float32),
        pltpu.VMEM((2, 128), jnp.int32),
        pltpu.VMEM((2, 128), jnp.float32),
        pltpu.VMEM((_ROWS_PER_W,), jnp.float32),
        pltpu.SemaphoreType.DMA,
        pltpu.SemaphoreType.DMA,
    ],
)


def kernel(input, keys, value):
    nq = jnp.linalg.norm(input, axis=-1, keepdims=True) + 1e-8
    nk = jnp.linalg.norm(keys, axis=-1, keepdims=True) + 1e-8
    keys_pad = jnp.pad(keys, ((0, _NPAD - _N), (0, 0)))
    nk_pad = jnp.pad(nk, ((0, _NPAD - _N), (0, 0)), constant_values=1.0)
    sims = _sims_call(input, nq, keys_pad, nk_pad)
    topk_sims, topk_idx, y = _sc_select(sims, value)
    return (y, topk_sims.reshape(_B, _CHOOSE), topk_idx.reshape(_B, _CHOOSE))


# speedup vs baseline: 4.1914x; 1.3845x over previous
import functools

import jax
import jax.numpy as jnp
from jax import lax
from jax.experimental import pallas as pl
from jax.experimental.pallas import tpu as pltpu
from jax.experimental.pallas import tpu_sc as plsc

_B = 1024
_K = 256
_N = 100000
_CHOOSE = 256
_INV_TEMP = 40.0


_NB = 1024
_NPAD = ((_N + _NB - 1) // _NB) * _NB
_SLAB = _NPAD // 128


def _mm_body(q_ref, nq_ref, k_ref, nk_ref, out_ref, qn_ref):
    i = pl.program_id(0)

    @pl.when(i == 0)
    def _():
        qn_ref[...] = q_ref[...] / nq_ref[...]

    kn = k_ref[...] / nk_ref[...]
    for t in range(_NB // 128):
        out_ref[:, t, :] = jax.lax.dot_general(
            qn_ref[...], kn[t * 128:(t + 1) * 128, :],
            (((1,), (1,)), ((), ())),
            preferred_element_type=jnp.float32,
            precision=jax.lax.Precision.DEFAULT)


_sims_call = pl.pallas_call(
    _mm_body,
    grid=(_NPAD // _NB,),
    in_specs=[
        pl.BlockSpec((_B, _K), lambda i: (0, 0)),
        pl.BlockSpec((_B, 1), lambda i: (0, 0)),
        pl.BlockSpec((_NB, _K), lambda i: (i, 0)),
        pl.BlockSpec((_NB, 1), lambda i: (i, 0)),
    ],
    out_specs=pl.BlockSpec((_B, _NB // 128, 128), lambda i: (0, i, 0)),
    out_shape=jax.ShapeDtypeStruct((_B, _SLAB, 128), jnp.float32),
    scratch_shapes=[pltpu.VMEM((_B, _K), jnp.float32)],
)


_L = 16
_NW = 32
_ROWS_PER_W = _B // _NW
_VREGS_ROW = _N // _L
_NBINS = 1024
_LO = -1.03125
_SCALE = 496.0
_SSTEP = 16
_SRANK = 36
_CAP = 1024
_PLANE = _CAP // _L
_CVR = _CAP // _L


def _sc_body(sims_hbm, value_hbm, out_sims, out_idx, out_y,
             row_v, hist_v, binsum_v, cand_val, cand_idx, osims_v, oidx_v,
             vals_v, ybuf, swp_val, swp_idx, sem, sem_g):
    wid = lax.axis_index("s") * 2 + lax.axis_index("c")
    lanes = lax.iota(jnp.int32, _L)
    lane_base = lanes * (_NBINS + 1)
    ones_i = jnp.ones((_L,), jnp.int32)
    zeros_i = jnp.zeros((_L,), jnp.int32)
    neg2 = jnp.full((_L,), -2.0, jnp.float32)
    laneoff = lanes * _PLANE
    plane_v = jnp.full((_L,), _PLANE, jnp.int32)

    def rload(v):
        s = v // 8
        return row_v[s, pl.ds((v - s * 8) * _L, _L)]

    def cval(v):
        return cand_val[pl.ds(v * _L, _L)]

    def cidx(v):
        return cand_idx[pl.ds(v * _L, _L)]

    def _hist_tf(step, rank, unroll):
        @plsc.parallel_loop(0, _NBINS + 1, unroll=8)
        def _zh(i):
            hist_v[pl.ds(i * _L, _L)] = zeros_i

        @plsc.parallel_loop(0, _VREGS_ROW, step=step, unroll=unroll)
        def _pa(j):
            v = rload(j)
            b = (v * _SCALE + (0.5 - _LO * _SCALE)).astype(jnp.int32)
            plsc.addupdate_scatter(hist_v, [lane_base + b], ones_i)

        @pl.loop(0, _NBINS // _L, unroll=2)
        def _bs(c):
            acc = hist_v[pl.ds(c * _L, _L)]
            for l in range(1, _L):
                acc = acc + hist_v[pl.ds(l * (_NBINS + 1) + c * _L, _L)]
            binsum_v[pl.ds(c * _L, _L)] = acc

        def _chunk_tot(c):
            return jnp.sum(binsum_v[pl.ds(c * _L, _L)])

        def _wcond(carry):
            cum, c = carry
            return jnp.logical_and(c > 0, cum + _chunk_tot(c) < rank)

        def _wstep(carry):
            cum, c = carry
            return cum + _chunk_tot(c), c - 1

        cum, cstar = lax.while_loop(
            _wcond, _wstep, (jnp.int32(0), jnp.int32(_NBINS // _L - 1)))
        sfx = cum + plsc.cumsum(
            lax.rev(binsum_v[pl.ds(cstar * _L, _L)], (0,)))
        i = jnp.max(plsc.all_reduce_ffs(sfx >= rank))
        bstar = cstar * _L + (_L - 1) - i
        tf = (bstar.astype(jnp.float32) - jnp.float32(0.5 - _LO * _SCALE)
              - 0.5) * jnp.float32(1.0 / _SCALE)
        return jnp.broadcast_to(tf, (_L,))

    def _collect(tfv):
        @plsc.parallel_loop(0, _CVR, unroll=8)
        def _zc(i):
            cand_val[pl.ds(i * _L, _L)] = neg2
            cand_idx[pl.ds(i * _L, _L)] = zeros_i

        @plsc.parallel_loop(0, _VREGS_ROW, unroll=8, carry=(zeros_i, lanes))
        def _pb(j, carry):
            cnt, jvec = carry
            v = rload(j)
            m = v >= tfv
            g = jnp.logical_and(m, cnt < plane_v)
            pos = laneoff + cnt
            plsc.store_scatter(cand_val, [pos], v, mask=g)
            plsc.store_scatter(cand_idx, [pos], jvec, mask=g)
            return cnt + m.astype(jnp.int32), jvec + _L

        return _pb[0]

    pltpu.async_copy(sims_hbm.at[wid * _ROWS_PER_W], row_v, sem)

    @pl.loop(0, _ROWS_PER_W)
    def _row(rl):
        r = wid * _ROWS_PER_W + rl
        pltpu.make_async_copy(sims_hbm.at[r], row_v, sem).wait()

        sc0 = jax.named_scope("phA")
        sc0.__enter__()
        tfv = _hist_tf(_SSTEP, _SRANK, 4)
        sc0.__exit__(None, None, None)
        sc2 = jax.named_scope("phB")
        sc2.__enter__()
        cnt = _collect(tfv)
        bad = jnp.logical_or(jnp.sum(jnp.minimum(cnt, plane_v)) < _CHOOSE,
                             jnp.max(cnt) > _PLANE)
        sc2.__exit__(None, None, None)
        sc5 = jax.named_scope("phFB")
        sc5.__enter__()

        @pl.when(bad)
        def _fallback():
            _collect(_hist_tf(1, _CHOOSE, 8))

        sc5.__exit__(None, None, None)
        sc3 = jax.named_scope("phSort")
        sc3.__enter__()
        @pl.when(rl + 1 < _ROWS_PER_W)
        def _pref():
            pltpu.async_copy(sims_hbm.at[r + 1], row_v, sem)

        def _ce(a, b, kv):
            ka = cval(a)
            kb = cval(b)
            ia = cidx(a)
            ib = cidx(b)
            desc = jnp.broadcast_to((a & kv) == 0, (_L,))
            swap = jnp.where(desc, ka < kb, ka > kb)
            cand_val[pl.ds(a * _L, _L)] = jnp.where(swap, kb, ka)
            cand_val[pl.ds(b * _L, _L)] = jnp.where(swap, ka, kb)
            cand_idx[pl.ds(a * _L, _L)] = jnp.where(swap, ib, ia)
            cand_idx[pl.ds(b * _L, _L)] = jnp.where(swap, ia, ib)

        def _vsort(v, desc):
            ks, xs = plsc.sort_key_val(cval(v), cidx(v), descending=desc)
            cand_val[pl.ds(v * _L, _L)] = ks
            cand_idx[pl.ds(v * _L, _L)] = xs

        @pl.loop(0, _CVR // 2)
        def _base(t):
            _vsort(2 * t, True)
            _vsort(2 * t + 1, False)

        kvs = []
        kv = 2
        while kv < _CVR:
            kvs.append(kv)
            kv *= 2
        for kv in kvs:
            jv = kv // 2
            while jv >= 1:
                @pl.loop(0, _CVR // 2)
                def _stage(t, jv=jv, kv=kv):
                    blk = t // jv
                    a = blk * (2 * jv) + (t - blk * jv)
                    _ce(a, a + jv, kv)
                jv //= 2

            @pl.loop(0, _CVR // 2)
            def _resort(t, kv=kv):
                blk = t // kv
                v = blk * (2 * kv) + (t - blk * kv)
                _vsort(v, True)
                _vsort(v + kv, False)

        @pl.loop(0, _CVR // 2)
        def _half1(t):
            _ce(t, t + _CVR // 2, _CVR)

        @pl.loop(0, _CVR // 4)
        def _half2(t):
            _ce(t, t + _CVR // 4, _CVR)

        jv = _CVR // 8
        while jv >= 1:
            @pl.loop(0, _CVR // 8)
            def _stage2(t, jv=jv):
                blk = t // jv
                a = blk * (2 * jv) + (t - blk * jv)
                _ce(a, a + jv, _CVR)
            jv //= 2

        @pl.loop(0, _CHOOSE // _L)
        def _resort_top(v):
            _vsort(v, True)

        xperm = jnp.bitwise_xor(lanes, 1)
        oddm = (lanes & 1) == 1

        def _pairfix(base):
            a = cand_val[pl.ds(base, _L)]
            ix = cand_idx[pl.ds(base, _L)]
            swp_val[...] = a
            swp_idx[...] = ix
            asw = plsc.load_gather(swp_val, [xperm])
            isw = plsc.load_gather(swp_idx, [xperm])
            cond = jnp.logical_and(a == asw, (ix > isw) != oddm)
            cand_idx[pl.ds(base, _L)] = jnp.where(cond, isw, ix)

        for par in (0, 1, 0):
            @pl.loop(0, _CHOOSE // _L)
            def _tp(t, par=par):
                _pairfix(t * _L + par)

        @pl.loop(0, _CHOOSE // _L)
        def _st(t):
            s = t // 8
            c = (t - s * 8) * _L
            osims_v[s, pl.ds(c, _L)] = cval(t)
            oidx_v[s, pl.ds(c, _L)] = cidx(t)

        sc3.__exit__(None, None, None)
        sc4 = jax.named_scope("phOut")
        sc4.__enter__()
        cp0 = pltpu.async_copy(value_hbm.at[oidx_v.at[0]], vals_v.at[0], sem_g)
        cp1 = pltpu.async_copy(value_hbm.at[oidx_v.at[1]], vals_v.at[1], sem_g)
        pltpu.sync_copy(osims_v, out_sims.at[r])
        pltpu.sync_copy(oidx_v, out_idx.at[r])
        cp0.wait()
        cp1.wait()

        mx = jnp.max(cval(0))

        def _sm(t, carry):
            accn, accd = carry
            s = t // 8
            c = (t - s * 8) * _L
            e = jnp.exp((osims_v[s, pl.ds(c, _L)] - mx) * _INV_TEMP)
            return accn + e * vals_v[s, pl.ds(c, _L)], accd + e

        accn, accd = lax.fori_loop(
            0, _CHOOSE // _L, _sm,
            (jnp.zeros((_L,), jnp.float32), jnp.zeros((_L,), jnp.float32)))
        yv = (jnp.broadcast_to(jnp.sum(accn), (_L,))
              / jnp.broadcast_to(jnp.sum(accd), (_L,)))
        plsc.store_scatter(ybuf, [jnp.broadcast_to(rl, (_L,))],
                           yv, mask=lanes < 1)

        sc4.__exit__(None, None, None)

    pltpu.sync_copy(ybuf, out_y.at[pl.ds(wid * _ROWS_PER_W, _ROWS_PER_W)])


_sc_select = pl.kernel(
    _sc_body,
    out_type=(
        jax.ShapeDtypeStruct((_B, 2, 128), jnp.float32),
        jax.ShapeDtypeStruct((_B, 2, 128), jnp.int32),
        jax.ShapeDtypeStruct((_B,), jnp.float32),
    ),
    mesh=plsc.VectorSubcoreMesh(core_axis_name="c", subcore_axis_name="s"),
    compiler_params=pltpu.CompilerParams(needs_layout_passes=False),
    scratch_types=[
        pltpu.VMEM((_SLAB, 128), jnp.float32),
        pltpu.VMEM(((_NBINS + 1) * _L,), jnp.int32),
        pltpu.VMEM((_NBINS,), jnp.int32),
        pltpu.VMEM((_CAP,), jnp.float32),
        pltpu.VMEM((_CAP,), jnp.int32),
        pltpu.VMEM((2, 128), jnp.float32),
        pltpu.VMEM((2, 128), jnp.int32),
        pltpu.VMEM((2, 128), jnp.float32),
        pltpu.VMEM((_ROWS_PER_W,), jnp.float32),
        pltpu.VMEM((_L,), jnp.float32),
        pltpu.VMEM((_L,), jnp.int32),
        pltpu.SemaphoreType.DMA,
        pltpu.SemaphoreType.DMA,
    ],
)


def kernel(input, keys, value):
    nq = jnp.linalg.norm(input, axis=-1, keepdims=True) + 1e-8
    nk = jnp.linalg.norm(keys, axis=-1, keepdims=True) + 1e-8
    keys_pad = jnp.pad(keys, ((0, _NPAD - _N), (0, 0)))
    nk_pad = jnp.pad(nk, ((0, _NPAD - _N), (0, 0)), constant_values=1.0)
    sims = _sims_call(input, nq, keys_pad, nk_pad)
    topk_sims, topk_idx, y = _sc_select(sims, value)
    return (y, topk_sims.reshape(_B, _CHOOSE), topk_idx.reshape(_B, _CHOOSE))


# speedup vs baseline: 4.2860x; 1.0226x over previous
import functools

import jax
import jax.numpy as jnp
from jax import lax
from jax.experimental import pallas as pl
from jax.experimental.pallas import tpu as pltpu
from jax.experimental.pallas import tpu_sc as plsc

_B = 1024
_K = 256
_N = 100000
_CHOOSE = 256
_INV_TEMP = 40.0


_NB = 1024
_NPAD = ((_N + _NB - 1) // _NB) * _NB
_SLAB = _NPAD // 128


def _mm_body(q_ref, nq_ref, k_ref, nk_ref, out_ref, qn_ref):
    i = pl.program_id(0)

    @pl.when(i == 0)
    def _():
        qn_ref[...] = q_ref[...] / nq_ref[...]

    kn = k_ref[...] / nk_ref[...]
    for t in range(_NB // 128):
        out_ref[:, t, :] = jax.lax.dot_general(
            qn_ref[...], kn[t * 128:(t + 1) * 128, :],
            (((1,), (1,)), ((), ())),
            preferred_element_type=jnp.float32,
            precision=jax.lax.Precision.DEFAULT)


_sims_call = pl.pallas_call(
    _mm_body,
    grid=(_NPAD // _NB,),
    in_specs=[
        pl.BlockSpec((_B, _K), lambda i: (0, 0)),
        pl.BlockSpec((_B, 1), lambda i: (0, 0)),
        pl.BlockSpec((_NB, _K), lambda i: (i, 0)),
        pl.BlockSpec((_NB, 1), lambda i: (i, 0)),
    ],
    out_specs=pl.BlockSpec((_B, _NB // 128, 128), lambda i: (0, i, 0)),
    out_shape=jax.ShapeDtypeStruct((_B, _SLAB, 128), jnp.float32),
    scratch_shapes=[pltpu.VMEM((_B, _K), jnp.float32)],
)


_L = 16
_NW = 32
_ROWS_PER_W = _B // _NW
_VREGS_ROW = _N // _L
_NBINS = 1024
_LO = -1.03125
_SCALE = 496.0
_SRANK = 51
_CVJ = 625
_CW = _CVJ * _L
_NCH = _VREGS_ROW // _CVJ
_NBUF = 5
_CAP = 1024
_PLANE = _CAP // _L
_CVR = _CAP // _L


def _sc_body(sims_hbm, value_hbm, out_sims, out_idx, out_y,
             rbuf, hist_v, binsum_v, cand_val, cand_idx, osims_v, oidx_v,
             vals_v, ybuf, swp_val, swp_idx,
             sem0, sem1, sem2, sem3, sem4, sem_g):
    sems = [sem0, sem1, sem2, sem3, sem4]
    wid = lax.axis_index("s") * 2 + lax.axis_index("c")
    lanes = lax.iota(jnp.int32, _L)
    lane_base = lanes * (_NBINS + 1)
    ones_i = jnp.ones((_L,), jnp.int32)
    zeros_i = jnp.zeros((_L,), jnp.int32)
    neg2 = jnp.full((_L,), -2.0, jnp.float32)
    laneoff = lanes * _PLANE
    plane_v = jnp.full((_L,), _PLANE, jnp.int32)

    def chunk_src(r, c):
        return sims_hbm.at[pl.ds(r * _NPAD + c * _CW, _CW)]

    def slot_ref(s):
        return rbuf.at[pl.ds(s * _CW, _CW)]

    def issue(r, c, s):
        pltpu.async_copy(chunk_src(r, c), slot_ref(s), sems[s])

    def wait_slot(s):
        pltpu.make_async_copy(chunk_src(0, 0), slot_ref(s), sems[s]).wait()

    def cload(s, j):
        return rbuf[pl.ds(s * _CW + j * _L, _L)]

    def cval(v):
        return cand_val[pl.ds(v * _L, _L)]

    def cidx(v):
        return cand_idx[pl.ds(v * _L, _L)]

    def _zero_hist():
        @plsc.parallel_loop(0, _NBINS + 1, unroll=8)
        def _zh(i):
            hist_v[pl.ds(i * _L, _L)] = zeros_i

    def _hist_chunk(s, unroll=8):
        @plsc.parallel_loop(0, _CVJ, unroll=unroll)
        def _pa(j):
            v = cload(s, j)
            b = (v * _SCALE + (0.5 - _LO * _SCALE)).astype(jnp.int32)
            plsc.addupdate_scatter(hist_v, [lane_base + b], ones_i)

    def _scan_tf(rank):
        @pl.loop(0, _NBINS // _L, unroll=2)
        def _bs(c):
            acc = hist_v[pl.ds(c * _L, _L)]
            for l in range(1, _L):
                acc = acc + hist_v[pl.ds(l * (_NBINS + 1) + c * _L, _L)]
            binsum_v[pl.ds(c * _L, _L)] = acc

        def _chunk_tot(c):
            return jnp.sum(binsum_v[pl.ds(c * _L, _L)])

        def _wcond(carry):
            cum, c = carry
            return jnp.logical_and(c > 0, cum + _chunk_tot(c) < rank)

        def _wstep(carry):
            cum, c = carry
            return cum + _chunk_tot(c), c - 1

        cum, cstar = lax.while_loop(
            _wcond, _wstep, (jnp.int32(0), jnp.int32(_NBINS // _L - 1)))
        sfx = cum + plsc.cumsum(
            lax.rev(binsum_v[pl.ds(cstar * _L, _L)], (0,)))
        i = jnp.max(plsc.all_reduce_ffs(sfx >= rank))
        bstar = cstar * _L + (_L - 1) - i
        tf = (bstar.astype(jnp.float32) - jnp.float32(0.5 - _LO * _SCALE)
              - 0.5) * jnp.float32(1.0 / _SCALE)
        return jnp.broadcast_to(tf, (_L,))

    def _zero_cand():
        @plsc.parallel_loop(0, _CVR, unroll=8)
        def _zc(i):
            cand_val[pl.ds(i * _L, _L)] = neg2
            cand_idx[pl.ds(i * _L, _L)] = zeros_i

    def _collect_chunk(s, jbase, cnt, tfv, unroll=8):
        @plsc.parallel_loop(0, _CVJ, unroll=unroll,
                            carry=(cnt, lanes + jbase))
        def _pb(j, carry):
            cnt, jvec = carry
            v = cload(s, j)
            m = v >= tfv
            g = jnp.logical_and(m, cnt < plane_v)
            pos = laneoff + cnt
            plsc.store_scatter(cand_val, [pos], v, mask=g)
            plsc.store_scatter(cand_idx, [pos], jvec, mask=g)
            return cnt + m.astype(jnp.int32), jvec + _L

        return _pb[0]

    r0 = wid * _ROWS_PER_W
    for c in range(_NBUF - 1):
        issue(r0, c, c)

    @pl.loop(0, _ROWS_PER_W)
    def _row(rl):
        r = r0 + rl

        wait_slot(0)
        _zero_hist()
        _hist_chunk(0)
        tfv = _scan_tf(_SRANK)

        _zero_cand()
        cnt = zeros_i
        for c in range(_NCH):
            if c > 0:
                wait_slot(c % _NBUF)
            nc = c + _NBUF - 1
            if nc < _NCH:
                issue(r, nc, nc % _NBUF)
            else:
                @pl.when(rl + 1 < _ROWS_PER_W)
                def _pref(nc=nc):
                    issue(r + 1, nc - _NCH, nc % _NBUF)
            cnt = _collect_chunk(c % _NBUF, c * _CW, cnt, tfv)

        bad = jnp.logical_or(jnp.sum(jnp.minimum(cnt, plane_v)) < _CHOOSE,
                             jnp.max(cnt) > _PLANE)

        @pl.when(bad)
        def _fallback():
            _zero_hist()

            @pl.loop(0, _NCH)
            def _fh(c):
                pltpu.sync_copy(chunk_src(r, c), slot_ref(_NBUF - 1))
                _hist_chunk(_NBUF - 1, unroll=2)

            tfx = _scan_tf(_CHOOSE)
            _zero_cand()

            def _fc(c, cc):
                pltpu.sync_copy(chunk_src(r, c), slot_ref(_NBUF - 1))
                return _collect_chunk(_NBUF - 1, c * _CW, cc, tfx, unroll=2)

            lax.fori_loop(0, _NCH, _fc, zeros_i)

        def _ce(a, b, kv):
            ka = cval(a)
            kb = cval(b)
            ia = cidx(a)
            ib = cidx(b)
            desc = jnp.broadcast_to((a & kv) == 0, (_L,))
            swap = jnp.where(desc, ka < kb, ka > kb)
            cand_val[pl.ds(a * _L, _L)] = jnp.where(swap, kb, ka)
            cand_val[pl.ds(b * _L, _L)] = jnp.where(swap, ka, kb)
            cand_idx[pl.ds(a * _L, _L)] = jnp.where(swap, ib, ia)
            cand_idx[pl.ds(b * _L, _L)] = jnp.where(swap, ia, ib)

        def _vsort(v, desc):
            ks, xs = plsc.sort_key_val(cval(v), cidx(v), descending=desc)
            cand_val[pl.ds(v * _L, _L)] = ks
            cand_idx[pl.ds(v * _L, _L)] = xs

        @pl.loop(0, _CVR // 2)
        def _base(t):
            _vsort(2 * t, True)
            _vsort(2 * t + 1, False)

        kvs = []
        kv = 2
        while kv < _CVR:
            kvs.append(kv)
            kv *= 2
        for kv in kvs:
            jv = kv // 2
            while jv >= 1:
                @pl.loop(0, _CVR // 2)
                def _stage(t, jv=jv, kv=kv):
                    blk = t // jv
                    a = blk * (2 * jv) + (t - blk * jv)
                    _ce(a, a + jv, kv)
                jv //= 2

            @pl.loop(0, _CVR // 2)
            def _resort(t, kv=kv):
                blk = t // kv
                v = blk * (2 * kv) + (t - blk * kv)
                _vsort(v, True)
                _vsort(v + kv, False)

        @pl.loop(0, _CVR // 2)
        def _half1(t):
            _ce(t, t + _CVR // 2, _CVR)

        @pl.loop(0, _CVR // 4)
        def _half2(t):
            _ce(t, t + _CVR // 4, _CVR)

        jv = _CVR // 8
        while jv >= 1:
            @pl.loop(0, _CVR // 8)
            def _stage2(t, jv=jv):
                blk = t // jv
                a = blk * (2 * jv) + (t - blk * jv)
                _ce(a, a + jv, _CVR)
            jv //= 2

        @pl.loop(0, _CHOOSE // _L)
        def _resort_top(v):
            _vsort(v, True)

        xperm = jnp.bitwise_xor(lanes, 1)
        oddm = (lanes & 1) == 1

        def _pairfix(base):
            a = cand_val[pl.ds(base, _L)]
            ix = cand_idx[pl.ds(base, _L)]
            swp_val[...] = a
            swp_idx[...] = ix
            asw = plsc.load_gather(swp_val, [xperm])
            isw = plsc.load_gather(swp_idx, [xperm])
            cond = jnp.logical_and(a == asw, (ix > isw) != oddm)
            cand_idx[pl.ds(base, _L)] = jnp.where(cond, isw, ix)

        for par in (0, 1, 0):
            @pl.loop(0, _CHOOSE // _L)
            def _tp(t, par=par):
                _pairfix(t * _L + par)

        @pl.loop(0, _CHOOSE // _L)
        def _st(t):
            s = t // 8
            c = (t - s * 8) * _L
            osims_v[s, pl.ds(c, _L)] = cval(t)
            oidx_v[s, pl.ds(c, _L)] = cidx(t)

        cp0 = pltpu.async_copy(value_hbm.at[oidx_v.at[0]], vals_v.at[0], sem_g)
        cp1 = pltpu.async_copy(value_hbm.at[oidx_v.at[1]], vals_v.at[1], sem_g)
        pltpu.sync_copy(osims_v, out_sims.at[r])
        pltpu.sync_copy(oidx_v, out_idx.at[r])
        cp0.wait()
        cp1.wait()

        mx = jnp.max(cval(0))

        def _sm(t, carry):
            accn, accd = carry
            s = t // 8
            c = (t - s * 8) * _L
            e = jnp.exp((osims_v[s, pl.ds(c, _L)] - mx) * _INV_TEMP)
            return accn + e * vals_v[s, pl.ds(c, _L)], accd + e

        accn, accd = lax.fori_loop(
            0, _CHOOSE // _L, _sm,
            (jnp.zeros((_L,), jnp.float32), jnp.zeros((_L,), jnp.float32)))
        yv = (jnp.broadcast_to(jnp.sum(accn), (_L,))
              / jnp.broadcast_to(jnp.sum(accd), (_L,)))
        plsc.store_scatter(ybuf, [jnp.broadcast_to(rl, (_L,))],
                           yv, mask=lanes < 1)

    pltpu.sync_copy(ybuf, out_y.at[pl.ds(wid * _ROWS_PER_W, _ROWS_PER_W)])


_sc_select = pl.kernel(
    _sc_body,
    out_type=(
        jax.ShapeDtypeStruct((_B, 2, 128), jnp.float32),
        jax.ShapeDtypeStruct((_B, 2, 128), jnp.int32),
        jax.ShapeDtypeStruct((_B,), jnp.float32),
    ),
    mesh=plsc.VectorSubcoreMesh(core_axis_name="c", subcore_axis_name="s"),
    compiler_params=pltpu.CompilerParams(needs_layout_passes=False),
    scratch_types=[
        pltpu.VMEM((_NBUF * _CW,), jnp.float32),
        pltpu.VMEM(((_NBINS + 1) * _L,), jnp.int32),
        pltpu.VMEM((_NBINS,), jnp.int32),
        pltpu.VMEM((_CAP,), jnp.float32),
        pltpu.VMEM((_CAP,), jnp.int32),
        pltpu.VMEM((2, 128), jnp.float32),
        pltpu.VMEM((2, 128), jnp.int32),
        pltpu.VMEM((2, 128), jnp.float32),
        pltpu.VMEM((_ROWS_PER_W,), jnp.float32),
        pltpu.VMEM((_L,), jnp.float32),
        pltpu.VMEM((_L,), jnp.int32),
        pltpu.SemaphoreType.DMA,
        pltpu.SemaphoreType.DMA,
        pltpu.SemaphoreType.DMA,
        pltpu.SemaphoreType.DMA,
        pltpu.SemaphoreType.DMA,
        pltpu.SemaphoreType.DMA,
    ],
)


def kernel(input, keys, value):
    nq = jnp.linalg.norm(input, axis=-1, keepdims=True) + 1e-8
    nk = jnp.linalg.norm(keys, axis=-1, keepdims=True) + 1e-8
    keys_pad = jnp.pad(keys, ((0, _NPAD - _N), (0, 0)))
    nk_pad = jnp.pad(nk, ((0, _NPAD - _N), (0, 0)), constant_values=1.0)
    sims = _sims_call(input, nq, keys_pad, nk_pad)
    topk_sims, topk_idx, y = _sc_select(sims.reshape(_B * _SLAB * 128), value)
    return (y, topk_sims.reshape(_B, _CHOOSE), topk_idx.reshape(_B, _CHOOSE))


# speedup vs baseline: 4.3620x; 1.0177x over previous
import functools

import jax
import jax.numpy as jnp
from jax import lax
from jax.experimental import pallas as pl
from jax.experimental.pallas import tpu as pltpu
from jax.experimental.pallas import tpu_sc as plsc

_B = 1024
_K = 256
_N = 100000
_CHOOSE = 256
_INV_TEMP = 40.0


_NB = 1024
_NPAD = ((_N + _NB - 1) // _NB) * _NB
_SLAB = _NPAD // 128


def _mm_body(q_ref, nq_ref, k_ref, nk_ref, out_ref, qn_ref):
    i = pl.program_id(0)

    @pl.when(i == 0)
    def _():
        qn_ref[...] = q_ref[...] / nq_ref[...]

    kn = k_ref[...] / nk_ref[...]
    for t in range(_NB // 128):
        out_ref[:, t, :] = jax.lax.dot_general(
            qn_ref[...], kn[t * 128:(t + 1) * 128, :],
            (((1,), (1,)), ((), ())),
            preferred_element_type=jnp.float32,
            precision=jax.lax.Precision.DEFAULT)


def _make_sims_call(nb_rows):
    return pl.pallas_call(
        _mm_body,
        grid=(_NPAD // _NB,),
        in_specs=[
            pl.BlockSpec((nb_rows, _K), lambda i: (0, 0)),
            pl.BlockSpec((nb_rows, 1), lambda i: (0, 0)),
            pl.BlockSpec((_NB, _K), lambda i: (i, 0)),
            pl.BlockSpec((_NB, 1), lambda i: (i, 0)),
        ],
        out_specs=pl.BlockSpec((nb_rows, _NB // 128, 128),
                               lambda i: (0, i, 0)),
        out_shape=jax.ShapeDtypeStruct((nb_rows, _SLAB, 128), jnp.float32),
        scratch_shapes=[pltpu.VMEM((nb_rows, _K), jnp.float32)],
    )


_L = 16
_NW = 32
_ROWS_PER_W = _B // _NW
_VREGS_ROW = _N // _L
_NBINS = 1024
_LO = -1.03125
_SCALE = 496.0
_SRANK = 51
_CVJ = 625
_CW = _CVJ * _L
_NCH = _VREGS_ROW // _CVJ
_NBUF = 5
_CAP = 1024
_PLANE = _CAP // _L
_CVR = _CAP // _L


def _sc_body(rows_per_w, sims_hbm, value_hbm, out_sims, out_idx, out_y,
             rbuf, hist_v, binsum_v, cand_val, cand_idx, osims_v, oidx_v,
             vals_v, ybuf, swp_val, swp_idx,
             sem0, sem1, sem2, sem3, sem4, sem_g):
    sems = [sem0, sem1, sem2, sem3, sem4]
    wid = lax.axis_index("s") * 2 + lax.axis_index("c")
    lanes = lax.iota(jnp.int32, _L)
    lane_base = lanes * (_NBINS + 1)
    ones_i = jnp.ones((_L,), jnp.int32)
    zeros_i = jnp.zeros((_L,), jnp.int32)
    neg2 = jnp.full((_L,), -2.0, jnp.float32)
    laneoff = lanes * _PLANE
    plane_v = jnp.full((_L,), _PLANE, jnp.int32)

    def chunk_src(r, c):
        return sims_hbm.at[pl.ds(r * _NPAD + c * _CW, _CW)]

    def slot_ref(s):
        return rbuf.at[pl.ds(s * _CW, _CW)]

    def issue(r, c, s):
        pltpu.async_copy(chunk_src(r, c), slot_ref(s), sems[s])

    def wait_slot(s):
        pltpu.make_async_copy(chunk_src(0, 0), slot_ref(s), sems[s]).wait()

    def cload(s, j):
        return rbuf[pl.ds(s * _CW + j * _L, _L)]

    def cval(v):
        return cand_val[pl.ds(v * _L, _L)]

    def cidx(v):
        return cand_idx[pl.ds(v * _L, _L)]

    def _zero_hist():
        @plsc.parallel_loop(0, _NBINS + 1, unroll=8)
        def _zh(i):
            hist_v[pl.ds(i * _L, _L)] = zeros_i

    def _hist_chunk(s, unroll=8):
        @plsc.parallel_loop(0, _CVJ, unroll=unroll)
        def _pa(j):
            v = cload(s, j)
            b = (v * _SCALE + (0.5 - _LO * _SCALE)).astype(jnp.int32)
            plsc.addupdate_scatter(hist_v, [lane_base + b], ones_i)

    def _scan_tf(rank):
        @pl.loop(0, _NBINS // _L, unroll=2)
        def _bs(c):
            acc = hist_v[pl.ds(c * _L, _L)]
            for l in range(1, _L):
                acc = acc + hist_v[pl.ds(l * (_NBINS + 1) + c * _L, _L)]
            binsum_v[pl.ds(c * _L, _L)] = acc

        def _chunk_tot(c):
            return jnp.sum(binsum_v[pl.ds(c * _L, _L)])

        def _wcond(carry):
            cum, c = carry
            return jnp.logical_and(c > 0, cum + _chunk_tot(c) < rank)

        def _wstep(carry):
            cum, c = carry
            return cum + _chunk_tot(c), c - 1

        cum, cstar = lax.while_loop(
            _wcond, _wstep, (jnp.int32(0), jnp.int32(_NBINS // _L - 1)))
        sfx = cum + plsc.cumsum(
            lax.rev(binsum_v[pl.ds(cstar * _L, _L)], (0,)))
        i = jnp.max(plsc.all_reduce_ffs(sfx >= rank))
        bstar = cstar * _L + (_L - 1) - i
        tf = (bstar.astype(jnp.float32) - jnp.float32(0.5 - _LO * _SCALE)
              - 0.5) * jnp.float32(1.0 / _SCALE)
        return jnp.broadcast_to(tf, (_L,))

    def _zero_cand():
        @plsc.parallel_loop(0, _CVR, unroll=8)
        def _zc(i):
            cand_val[pl.ds(i * _L, _L)] = neg2
            cand_idx[pl.ds(i * _L, _L)] = zeros_i

    def _collect_chunk(s, jbase, cnt, tfv, unroll=8):
        @plsc.parallel_loop(0, _CVJ, unroll=unroll,
                            carry=(cnt, lanes + jbase))
        def _pb(j, carry):
            cnt, jvec = carry
            v = cload(s, j)
            m = v >= tfv
            g = jnp.logical_and(m, cnt < plane_v)
            pos = laneoff + cnt
            plsc.store_scatter(cand_val, [pos], v, mask=g)
            plsc.store_scatter(cand_idx, [pos], jvec, mask=g)
            return cnt + m.astype(jnp.int32), jvec + _L

        return _pb[0]

    r0 = wid * rows_per_w
    for c in range(_NBUF - 1):
        issue(r0, c, c)

    @pl.loop(0, rows_per_w)
    def _row(rl):
        r = r0 + rl

        wait_slot(0)
        _zero_hist()
        _hist_chunk(0)
        tfv = _scan_tf(_SRANK)

        _zero_cand()
        cnt = zeros_i
        for c in range(_NCH):
            if c > 0:
                wait_slot(c % _NBUF)
            nc = c + _NBUF - 1
            if nc < _NCH:
                issue(r, nc, nc % _NBUF)
            else:
                @pl.when(rl + 1 < rows_per_w)
                def _pref(nc=nc):
                    issue(r + 1, nc - _NCH, nc % _NBUF)
            cnt = _collect_chunk(c % _NBUF, c * _CW, cnt, tfv)

        bad = jnp.logical_or(jnp.sum(jnp.minimum(cnt, plane_v)) < _CHOOSE,
                             jnp.max(cnt) > _PLANE)

        @pl.when(bad)
        def _fallback():
            _zero_hist()

            @pl.loop(0, _NCH)
            def _fh(c):
                pltpu.sync_copy(chunk_src(r, c), slot_ref(_NBUF - 1))
                _hist_chunk(_NBUF - 1, unroll=2)

            tfx = _scan_tf(_CHOOSE)
            _zero_cand()

            def _fc(c, cc):
                pltpu.sync_copy(chunk_src(r, c), slot_ref(_NBUF - 1))
                return _collect_chunk(_NBUF - 1, c * _CW, cc, tfx, unroll=2)

            lax.fori_loop(0, _NCH, _fc, zeros_i)

        def _ce(a, b, kv):
            ka = cval(a)
            kb = cval(b)
            ia = cidx(a)
            ib = cidx(b)
            desc = jnp.broadcast_to((a & kv) == 0, (_L,))
            swap = jnp.where(desc, ka < kb, ka > kb)
            cand_val[pl.ds(a * _L, _L)] = jnp.where(swap, kb, ka)
            cand_val[pl.ds(b * _L, _L)] = jnp.where(swap, ka, kb)
            cand_idx[pl.ds(a * _L, _L)] = jnp.where(swap, ib, ia)
            cand_idx[pl.ds(b * _L, _L)] = jnp.where(swap, ia, ib)

        def _vsort(v, desc):
            ks, xs = plsc.sort_key_val(cval(v), cidx(v), descending=desc)
            cand_val[pl.ds(v * _L, _L)] = ks
            cand_idx[pl.ds(v * _L, _L)] = xs

        @pl.loop(0, _CVR // 2)
        def _base(t):
            _vsort(2 * t, True)
            _vsort(2 * t + 1, False)

        kvs = []
        kv = 2
        while kv < _CVR:
            kvs.append(kv)
            kv *= 2
        for kv in kvs:
            jv = kv // 2
            while jv >= 1:
                @pl.loop(0, _CVR // 2)
                def _stage(t, jv=jv, kv=kv):
                    blk = t // jv
                    a = blk * (2 * jv) + (t - blk * jv)
                    _ce(a, a + jv, kv)
                jv //= 2

            @pl.loop(0, _CVR // 2)
            def _resort(t, kv=kv):
                blk = t // kv
                v = blk * (2 * kv) + (t - blk * kv)
                _vsort(v, True)
                _vsort(v + kv, False)

        @pl.loop(0, _CVR // 2)
        def _half1(t):
            _ce(t, t + _CVR // 2, _CVR)

        @pl.loop(0, _CVR // 4)
        def _half2(t):
            _ce(t, t + _CVR // 4, _CVR)

        jv = _CVR // 8
        while jv >= 1:
            @pl.loop(0, _CVR // 8)
            def _stage2(t, jv=jv):
                blk = t // jv
                a = blk * (2 * jv) + (t - blk * jv)
                _ce(a, a + jv, _CVR)
            jv //= 2

        @pl.loop(0, _CHOOSE // _L)
        def _resort_top(v):
            _vsort(v, True)

        xperm = jnp.bitwise_xor(lanes, 1)
        oddm = (lanes & 1) == 1

        def _pairfix(base):
            a = cand_val[pl.ds(base, _L)]
            ix = cand_idx[pl.ds(base, _L)]
            swp_val[...] = a
            swp_idx[...] = ix
            asw = plsc.load_gather(swp_val, [xperm])
            isw = plsc.load_gather(swp_idx, [xperm])
            cond = jnp.logical_and(a == asw, (ix > isw) != oddm)
            cand_idx[pl.ds(base, _L)] = jnp.where(cond, isw, ix)

        for par in (0, 1, 0):
            @pl.loop(0, _CHOOSE // _L)
            def _tp(t, par=par):
                _pairfix(t * _L + par)

        @pl.loop(0, _CHOOSE // _L)
        def _st(t):
            s = t // 8
            c = (t - s * 8) * _L
            osims_v[s, pl.ds(c, _L)] = cval(t)
            oidx_v[s, pl.ds(c, _L)] = cidx(t)

        cp0 = pltpu.async_copy(value_hbm.at[oidx_v.at[0]], vals_v.at[0], sem_g)
        cp1 = pltpu.async_copy(value_hbm.at[oidx_v.at[1]], vals_v.at[1], sem_g)
        pltpu.sync_copy(osims_v, out_sims.at[r])
        pltpu.sync_copy(oidx_v, out_idx.at[r])
        cp0.wait()
        cp1.wait()

        mx = jnp.max(cval(0))

        def _sm(t, carry):
            accn, accd = carry
            s = t // 8
            c = (t - s * 8) * _L
            e = jnp.exp((osims_v[s, pl.ds(c, _L)] - mx) * _INV_TEMP)
            return accn + e * vals_v[s, pl.ds(c, _L)], accd + e

        accn, accd = lax.fori_loop(
            0, _CHOOSE // _L, _sm,
            (jnp.zeros((_L,), jnp.float32), jnp.zeros((_L,), jnp.float32)))
        yv = (jnp.broadcast_to(jnp.sum(accn), (_L,))
              / jnp.broadcast_to(jnp.sum(accd), (_L,)))
        plsc.store_scatter(ybuf, [jnp.broadcast_to(rl, (_L,))],
                           yv, mask=lanes < 1)

    pltpu.sync_copy(ybuf, out_y.at[pl.ds(wid * rows_per_w, rows_per_w)])


def _make_sc_select(nrows):
    rows_per_w = nrows // _NW
    return pl.kernel(
        functools.partial(_sc_body, rows_per_w),
        out_type=(
            jax.ShapeDtypeStruct((nrows, 2, 128), jnp.float32),
            jax.ShapeDtypeStruct((nrows, 2, 128), jnp.int32),
            jax.ShapeDtypeStruct((nrows,), jnp.float32),
        ),
        mesh=plsc.VectorSubcoreMesh(core_axis_name="c", subcore_axis_name="s"),
        compiler_params=pltpu.CompilerParams(needs_layout_passes=False),
        scratch_types=[
            pltpu.VMEM((_NBUF * _CW,), jnp.float32),
            pltpu.VMEM(((_NBINS + 1) * _L,), jnp.int32),
            pltpu.VMEM((_NBINS,), jnp.int32),
            pltpu.VMEM((_CAP,), jnp.float32),
            pltpu.VMEM((_CAP,), jnp.int32),
            pltpu.VMEM((2, 128), jnp.float32),
            pltpu.VMEM((2, 128), jnp.int32),
            pltpu.VMEM((2, 128), jnp.float32),
            pltpu.VMEM((rows_per_w,), jnp.float32),
            pltpu.VMEM((_L,), jnp.float32),
            pltpu.VMEM((_L,), jnp.int32),
            pltpu.SemaphoreType.DMA,
            pltpu.SemaphoreType.DMA,
            pltpu.SemaphoreType.DMA,
            pltpu.SemaphoreType.DMA,
            pltpu.SemaphoreType.DMA,
            pltpu.SemaphoreType.DMA,
        ],
    )


_HB = _B // 2
_mm_half = _make_sims_call(_HB)
_sc_half = _make_sc_select(_HB)


def kernel(input, keys, value):
    nq = jnp.linalg.norm(input, axis=-1, keepdims=True) + 1e-8
    nk = jnp.linalg.norm(keys, axis=-1, keepdims=True) + 1e-8
    keys_pad = jnp.pad(keys, ((0, _NPAD - _N), (0, 0)))
    nk_pad = jnp.pad(nk, ((0, _NPAD - _N), (0, 0)), constant_values=1.0)
    outs = []
    for h in range(2):
        sl = slice(h * _HB, (h + 1) * _HB)
        sims = _mm_half(input[sl], nq[sl], keys_pad, nk_pad)
        outs.append(_sc_half(sims.reshape(_HB * _SLAB * 128), value))
    y = jnp.concatenate([outs[0][2], outs[1][2]])
    topk_sims = jnp.concatenate([outs[0][0], outs[1][0]]).reshape(_B, _CHOOSE)
    topk_idx = jnp.concatenate([outs[0][1], outs[1][1]]).reshape(_B, _CHOOSE)
    return (y, topk_sims, topk_idx)
